# Initial kernel scaffold; baseline (speedup 1.0000x reference)
import jax
import jax.numpy as jnp
from jax.experimental import pallas as pl


def kernel(dist, embed_q):
    raise NotImplementedError("write your pallas kernel here")


# speedup vs baseline: 6.3924x; 6.3924x over previous
import jax
import jax.numpy as jnp
from jax.experimental import pallas as pl
from jax.experimental.pallas import tpu as pltpu

_NUM_SLOTS = 21
_DIM = 64
_KPAD = 32
_BN = 2048


def _body(x_ref, e_ref, o_ref):
    t = jnp.clip(x_ref[...] * 20.0, 0.0, 20.0)
    k = jax.lax.broadcasted_iota(jnp.int32, (_BN, _KPAD), 1).astype(jnp.float32)
    w = jnp.maximum(1.0 - jnp.abs(t - k), 0.0)
    o_ref[...] = jnp.dot(w, e_ref[...], preferred_element_type=jnp.float32,
                         precision=jax.lax.Precision.HIGHEST)


def kernel(dist, embed_q):
    n = dist.shape[0] * dist.shape[1]
    x = dist.reshape(n, 1)
    e = jnp.zeros((_KPAD, _DIM), jnp.float32).at[:_NUM_SLOTS].set(embed_q)
    out = pl.pallas_call(
        _body,
        grid=(n // _BN,),
        in_specs=[
            pl.BlockSpec((_BN, 1), lambda i: (i, 0)),
            pl.BlockSpec((_KPAD, _DIM), lambda i: (0, 0)),
        ],
        out_specs=pl.BlockSpec((_BN, _DIM), lambda i: (i, 0)),
        out_shape=jax.ShapeDtypeStruct((n, _DIM), jnp.float32),
    )(x, e)
    return out.reshape(dist.shape[0], dist.shape[1], _DIM)


# speedup vs baseline: 7.3097x; 1.1435x over previous
import jax
import jax.numpy as jnp
from jax.experimental import pallas as pl
from jax.experimental.pallas import tpu as pltpu

_NUM_SLOTS = 21
_DIM = 64
_KPAD = 32
_BN = 2048


def _body(x_ref, e_ref, o_ref):
    t = jnp.clip(x_ref[...] * 20.0, 0.0, 20.0)
    k = jax.lax.broadcasted_iota(jnp.int32, (_BN, _KPAD), 1).astype(jnp.float32)
    w = jnp.maximum(1.0 - jnp.abs(t - k), 0.0)
    o_ref[...] = jnp.dot(w, e_ref[...], preferred_element_type=jnp.float32)


def kernel(dist, embed_q):
    n = dist.shape[0] * dist.shape[1]
    x = dist.reshape(n, 1)
    e = jnp.zeros((_KPAD, _DIM), jnp.float32).at[:_NUM_SLOTS].set(embed_q)
    out = pl.pallas_call(
        _body,
        grid=(n // _BN,),
        in_specs=[
            pl.BlockSpec((_BN, 1), lambda i: (i, 0)),
            pl.BlockSpec((_KPAD, _DIM), lambda i: (0, 0)),
        ],
        out_specs=pl.BlockSpec((_BN, _DIM), lambda i: (i, 0)),
        out_shape=jax.ShapeDtypeStruct((n, _DIM), jnp.float32),
    )(x, e)
    return out.reshape(dist.shape[0], dist.shape[1], _DIM)


# speedup vs baseline: 8.5789x; 1.1736x over previous
import jax
import jax.numpy as jnp
from jax.experimental import pallas as pl
from jax.experimental.pallas import tpu as pltpu

_NUM_SLOTS = 21
_DIM = 64
_KPAD = 32
_B0 = 8


def _body(x_ref, e_ref, o_ref):
    step = x_ref.shape[1]
    xt = x_ref[...].T
    chunks = []
    for b in range(_B0):
        t = jnp.clip(xt[:, b:b + 1] * 20.0, 0.0, 20.0)
        k = jax.lax.broadcasted_iota(jnp.int32, (step, _KPAD), 1)
        chunks.append(jnp.maximum(1.0 - jnp.abs(t - k.astype(jnp.float32)), 0.0))
    w = jnp.concatenate(chunks, axis=0)
    m = jnp.dot(w, e_ref[...], preferred_element_type=jnp.float32)
    o_ref[...] = m.reshape(_B0, step, _DIM)


def kernel(dist, embed_q):
    batch, step = dist.shape
    e = jnp.zeros((_KPAD, _DIM), jnp.float32).at[:_NUM_SLOTS].set(embed_q)
    return pl.pallas_call(
        _body,
        grid=(batch // _B0,),
        in_specs=[
            pl.BlockSpec((_B0, step), lambda i: (i, 0)),
            pl.BlockSpec((_KPAD, _DIM), lambda i: (0, 0)),
        ],
        out_specs=pl.BlockSpec((_B0, step, _DIM), lambda i: (i, 0, 0)),
        out_shape=jax.ShapeDtypeStruct((batch, step, _DIM), jnp.float32),
    )(dist, e)


# speedup vs baseline: 11.0037x; 1.2827x over previous
import jax
import jax.numpy as jnp
from jax.experimental import pallas as pl
from jax.experimental.pallas import tpu as pltpu

_NUM_SLOTS = 21
_DIM = 64
_KPAD = 32
_B0 = 16


def _body(x_ref, e_ref, o_ref):
    step = x_ref.shape[1]
    xt = x_ref[...].T
    chunks = []
    for b in range(_B0):
        t = jnp.clip(xt[:, b:b + 1] * 20.0, 0.0, 20.0)
        k = jax.lax.broadcasted_iota(jnp.int32, (step, _KPAD), 1)
        chunks.append(jnp.maximum(1.0 - jnp.abs(t - k.astype(jnp.float32)), 0.0))
    w = jnp.concatenate(chunks, axis=0)
    m = jnp.dot(w, e_ref[...], preferred_element_type=jnp.float32)
    o_ref[...] = m.reshape(_B0, step, _DIM)


def kernel(dist, embed_q):
    batch, step = dist.shape
    e = jnp.zeros((_KPAD, _DIM), jnp.float32).at[:_NUM_SLOTS].set(embed_q)
    return pl.pallas_call(
        _body,
        grid=(batch // _B0,),
        in_specs=[
            pl.BlockSpec((_B0, step), lambda i: (i, 0)),
            pl.BlockSpec((_KPAD, _DIM), lambda i: (0, 0)),
        ],
        out_specs=pl.BlockSpec((_B0, step, _DIM), lambda i: (i, 0, 0)),
        out_shape=jax.ShapeDtypeStruct((batch, step, _DIM), jnp.float32),
    )(dist, e)


# speedup vs baseline: 12.6954x; 1.1537x over previous
import jax
import jax.numpy as jnp
from jax.experimental import pallas as pl
from jax.experimental.pallas import tpu as pltpu

_NUM_SLOTS = 21
_DIM = 64
_KPAD = 32
_B0 = 32


def _body(x_ref, e_ref, o_ref):
    step = x_ref.shape[1]
    xt = x_ref[...].T
    chunks = []
    for b in range(_B0):
        t = jnp.clip(xt[:, b:b + 1] * 20.0, 0.0, 20.0)
        k = jax.lax.broadcasted_iota(jnp.int32, (step, _KPAD), 1)
        chunks.append(jnp.maximum(1.0 - jnp.abs(t - k.astype(jnp.float32)), 0.0))
    w = jnp.concatenate(chunks, axis=0)
    m = jnp.dot(w, e_ref[...], preferred_element_type=jnp.float32)
    o_ref[...] = m.reshape(_B0, step, _DIM)


def kernel(dist, embed_q):
    batch, step = dist.shape
    e = jnp.zeros((_KPAD, _DIM), jnp.float32).at[:_NUM_SLOTS].set(embed_q)
    return pl.pallas_call(
        _body,
        grid=(batch // _B0,),
        in_specs=[
            pl.BlockSpec((_B0, step), lambda i: (i, 0)),
            pl.BlockSpec((_KPAD, _DIM), lambda i: (0, 0)),
        ],
        out_specs=pl.BlockSpec((_B0, step, _DIM), lambda i: (i, 0, 0)),
        out_shape=jax.ShapeDtypeStruct((batch, step, _DIM), jnp.float32),
    )(dist, e)
